# baseline (speedup 1.0000x reference)
import jax
import jax.numpy as jnp
from jax import lax
from jax.experimental import pallas as pl
from jax.experimental.pallas import tpu as pltpu
from jax.experimental.pallas import tpu_sc as plsc

_R, _C = 128, 32768
_NC, _NS = 2, 16
_NW = _NC * _NS
_L = 16
_ROWS_PER_W = _R // _NW
_NV = _C // _L


def _lanes_min(acc):
    dnums = lax.GatherDimensionNumbers(
        offset_dims=(), collapsed_slice_dims=(0,), start_index_map=(0,)
    )
    for shift in (8, 4, 2, 1):
        idx = lax.rem(lax.iota(jnp.int32, _L) + shift, _L)
        rot = lax.gather(
            acc,
            idx[:, None],
            dnums,
            slice_sizes=(1,),
            mode=lax.GatherScatterMode.PROMISE_IN_BOUNDS,
        )
        acc = jnp.minimum(acc, rot)
    return acc


_UNROLL = 8
_NBUF = 3
_CHUNK_V = 64
_NCHUNK = _NV // _CHUNK_V


def _min_mask_row(buf, cmins):
    inf = jnp.full((_L,), jnp.inf, jnp.float32)

    def chunk_body(c, gacc):
        base = c * _CHUNK_V

        @plsc.parallel_loop(0, _CHUNK_V, step=_UNROLL, carry=(inf,) * _UNROLL)
        def cloop(i, accs):
            return tuple(
                jnp.minimum(a, buf[pl.ds((base + i + k) * _L, _L)])
                for k, a in enumerate(accs)
            )

        cmin = cloop[0]
        for a in cloop[1:]:
            cmin = jnp.minimum(cmin, a)
        cmins[pl.ds(c * _L, _L)] = cmin
        return jnp.minimum(gacc, cmin)

    gacc = lax.fori_loop(0, _NCHUNK, chunk_body, inf)
    thv = _lanes_min(gacc)

    def fix_body(c, carry):
        lm = _lanes_min(cmins[pl.ds(c * _L, _L)])
        hit = lm[0] <= thv[0]

        @pl.when(hit)
        def _():
            base = c * _CHUNK_V

            @plsc.parallel_loop(0, _CHUNK_V, step=1, unroll=_UNROLL)
            def floop(i):
                v = buf[pl.ds((base + i) * _L, _L)]
                buf[pl.ds((base + i) * _L, _L)] = jnp.where(v > thv, v, 0.0)

        return carry

    lax.fori_loop(0, _NCHUNK, fix_body, 0)


def _ash_body(x_hbm, out_hbm, *scratch):
    bufs = scratch[:_NBUF]
    cmins = scratch[_NBUF]
    lsems = scratch[_NBUF + 1:_NBUF + 1 + _NBUF]
    ssems = scratch[_NBUF + 1 + _NBUF:]
    wid = lax.axis_index("s") * _NC + lax.axis_index("c")
    base = wid * _ROWS_PER_W

    loads = [None] * _ROWS_PER_W
    stores = [None] * _ROWS_PER_W
    pending = set()
    for r in range(min(_NBUF, _ROWS_PER_W)):
        loads[r] = pltpu.async_copy(x_hbm.at[base + r], bufs[r % _NBUF], lsems[r % _NBUF])
    for r in range(_ROWS_PER_W):
        p = r % _NBUF
        nxt = r + 1
        if _NBUF <= nxt < _ROWS_PER_W:
            stores[nxt - _NBUF].wait()
            pending.discard(nxt - _NBUF)
            loads[nxt] = pltpu.async_copy(
                x_hbm.at[base + nxt], bufs[nxt % _NBUF], lsems[nxt % _NBUF]
            )
        loads[r].wait()
        _min_mask_row(bufs[p], cmins)
        stores[r] = pltpu.async_copy(bufs[p], out_hbm.at[base + r], ssems[p])
        pending.add(r)
    for r in sorted(pending):
        stores[r].wait()


def kernel(input, k_ash_):
    del k_ash_
    fn = pl.kernel(
        _ash_body,
        out_type=jax.ShapeDtypeStruct((_R, _C), jnp.float32),
        mesh=plsc.VectorSubcoreMesh(core_axis_name="c", subcore_axis_name="s"),
        scratch_types=(
            [pltpu.VMEM((_C,), jnp.float32)] * _NBUF
            + [pltpu.VMEM((_NCHUNK * _L,), jnp.float32)]
            + [pltpu.SemaphoreType.DMA] * (2 * _NBUF)
        ),
    )
    return fn(input)


# speedup vs baseline: 1.0674x; 1.0674x over previous
import jax
import jax.numpy as jnp
from jax import lax
from jax.experimental import pallas as pl
from jax.experimental.pallas import tpu as pltpu
from jax.experimental.pallas import tpu_sc as plsc

_R, _C = 128, 32768
_NC, _NS = 2, 16
_NW = _NC * _NS
_L = 16
_ROWS_PER_W = _R // _NW
_NV = _C // _L
_UNROLL = 8
_NBUF = 3
_CHUNK_V = 64
_NCHUNK = _NV // _CHUNK_V


def _lanes_min(acc):
    dnums = lax.GatherDimensionNumbers(
        offset_dims=(), collapsed_slice_dims=(0,), start_index_map=(0,)
    )
    for shift in (8, 4, 2, 1):
        idx = lax.rem(lax.iota(jnp.int32, _L) + shift, _L)
        rot = lax.gather(
            acc,
            idx[:, None],
            dnums,
            slice_sizes=(1,),
            mode=lax.GatherScatterMode.PROMISE_IN_BOUNDS,
        )
        acc = jnp.minimum(acc, rot)
    return acc


def _pass1(buf, cmins):
    inf = jnp.full((_L,), jnp.inf, jnp.float32)

    @plsc.parallel_loop(0, _NCHUNK)
    def chunk_loop(c):
        cbase = c * _CHUNK_V * _L
        accs = [inf] * _UNROLL
        for k in range(_CHUNK_V):
            accs[k % _UNROLL] = jnp.minimum(
                accs[k % _UNROLL], buf[pl.ds(cbase + k * _L, _L)]
            )
        m = accs[0]
        for a in accs[1:]:
            m = jnp.minimum(m, a)
        cmins[pl.ds(c * _L, _L)] = m

    g = cmins[pl.ds(0, _L)]
    for c in range(1, _NCHUNK):
        g = jnp.minimum(g, cmins[pl.ds(c * _L, _L)])
    return _lanes_min(g)


def _ash_body(x_hbm, out_hbm, *scratch):
    bufs = scratch[:_NBUF]
    cmins = scratch[_NBUF]
    fixbuf = scratch[_NBUF + 1]
    lsems = scratch[_NBUF + 2:_NBUF + 2 + _NBUF]
    ssems = scratch[_NBUF + 2 + _NBUF:_NBUF + 2 + 2 * _NBUF]
    fsem = scratch[_NBUF + 2 + 2 * _NBUF]
    wid = lax.axis_index("s") * _NC + lax.axis_index("c")
    base = wid * _ROWS_PER_W

    loads = [None] * _ROWS_PER_W
    for r in range(min(_NBUF, _ROWS_PER_W)):
        loads[r] = pltpu.async_copy(x_hbm.at[base + r], bufs[r % _NBUF], lsems[r % _NBUF])
    for r in range(_ROWS_PER_W):
        p = r % _NBUF
        buf = bufs[p]
        loads[r].wait()
        store = pltpu.async_copy(buf, out_hbm.at[base + r], ssems[p])
        thv = _pass1(buf, cmins)
        store.wait()

        def fix_body(c, carry):
            lm = _lanes_min(cmins[pl.ds(c * _L, _L)])
            hit = lm[0] <= thv[0]

            @pl.when(hit)
            def _():
                cbase = c * _CHUNK_V * _L

                @plsc.parallel_loop(0, _CHUNK_V, step=1, unroll=_UNROLL)
                def floop(i):
                    v = buf[pl.ds(cbase + i * _L, _L)]
                    fixbuf[pl.ds(i * _L, _L)] = jnp.where(v > thv, v, 0.0)

                pltpu.async_copy(
                    fixbuf,
                    out_hbm.at[base + r, pl.ds(cbase, _CHUNK_V * _L)],
                    fsem,
                ).wait()

            return carry

        lax.fori_loop(0, _NCHUNK, fix_body, 0)
        nxt = r + _NBUF
        if nxt < _ROWS_PER_W:
            loads[nxt] = pltpu.async_copy(x_hbm.at[base + nxt], bufs[p], lsems[p])


def kernel(input, k_ash_):
    del k_ash_
    fn = pl.kernel(
        _ash_body,
        out_type=jax.ShapeDtypeStruct((_R, _C), jnp.float32),
        mesh=plsc.VectorSubcoreMesh(core_axis_name="c", subcore_axis_name="s"),
        scratch_types=(
            [pltpu.VMEM((_C,), jnp.float32)] * _NBUF
            + [
                pltpu.VMEM((_NCHUNK * _L,), jnp.float32),
                pltpu.VMEM((_CHUNK_V * _L,), jnp.float32),
            ]
            + [pltpu.SemaphoreType.DMA] * (2 * _NBUF + 1)
        ),
    )
    return fn(input)
